# Initial kernel scaffold; baseline (speedup 1.0000x reference)
import jax
import jax.numpy as jnp
from jax.experimental import pallas as pl


def kernel(in_feat, edge_index, W1, b1, W2, b2):
    raise NotImplementedError("write your pallas kernel here")


# speedup vs baseline: 11.3033x; 11.3033x over previous
import functools

import jax
import jax.numpy as jnp
from jax import lax
from jax.experimental import pallas as pl
from jax.experimental.pallas import tpu as pltpu
from jax.experimental.pallas import tpu_sc as plsc

N = 10000
E = 160000
NPAD = 10240
F = 256
HALF = 128
NTILES = 16
EPT = E // NTILES
SUB = 80
NSUB = EPT // SUB
BLK = 1600
SPB = BLK // SUB
ROWS_PT = NPAD // NTILES

_mesh = plsc.VectorSubcoreMesh(core_axis_name="c", subcore_axis_name="s")


@functools.partial(
    pl.kernel,
    out_type=jax.ShapeDtypeStruct((2, NTILES, NPAD), jnp.float32),
    mesh=_mesh,
    scratch_types=[
        pltpu.VMEM((EPT,), jnp.int32),
        pltpu.VMEM((NPAD,), jnp.float32),
    ],
    compiler_params=pltpu.CompilerParams(needs_layout_passes=False),
)
def _sc_degrees(src_hbm, dst_hbm, out_hbm, idx_v, acc_v):
    c = lax.axis_index("c")
    s = lax.axis_index("s")

    @pl.when(c == 0)
    def _load_src():
        pltpu.sync_copy(src_hbm.at[pl.ds(s * EPT, EPT)], idx_v)

    @pl.when(c == 1)
    def _load_dst():
        pltpu.sync_copy(dst_hbm.at[pl.ds(s * EPT, EPT)], idx_v)

    def zero(i, carry):
        acc_v[pl.ds(i * 16, 16)] = jnp.zeros((16,), jnp.float32)
        return carry

    lax.fori_loop(0, NPAD // 16, zero, 0)

    ones = jnp.ones((16,), jnp.float32)

    def body(i, carry):
        u = idx_v[pl.ds(i * 16, 16)]
        plsc.addupdate_scatter(acc_v, [u], ones)
        return carry

    lax.fori_loop(0, EPT // 16, body, 0)
    pltpu.sync_copy(acc_v, out_hbm.at[c, s])


def _tc_scale_matmul_body(parts_ref, x_ref, w1_ref, y_ref, rin_ref, rout_ref):
    deg = jnp.sum(parts_ref[...], axis=1)
    r = lax.rsqrt(jnp.clip(deg, 1.0, None))
    rout = r[0]
    rin = r[1]
    y = jnp.dot(x_ref[...], w1_ref[...], preferred_element_type=jnp.float32)
    y_ref[...] = y * rout[:, None]
    rin_ref[...] = rin
    rout_ref[...] = rout


def _tc_scale_matmul(parts, x_pad, W1):
    BN = 512
    grid = (NPAD // BN,)
    return pl.pallas_call(
        _tc_scale_matmul_body,
        grid=grid,
        in_specs=[
            pl.BlockSpec((2, NTILES, BN), lambda i: (0, 0, i)),
            pl.BlockSpec((BN, F), lambda i: (i, 0)),
            pl.BlockSpec((F, F), lambda i: (0, 0)),
        ],
        out_specs=[
            pl.BlockSpec((BN, F), lambda i: (i, 0)),
            pl.BlockSpec((BN,), lambda i: (i,)),
            pl.BlockSpec((BN,), lambda i: (i,)),
        ],
        out_shape=[
            jax.ShapeDtypeStruct((NPAD, F), jnp.float32),
            jax.ShapeDtypeStruct((NPAD,), jnp.float32),
            jax.ShapeDtypeStruct((NPAD,), jnp.float32),
        ],
    )(parts, x_pad, W1)


@functools.partial(
    pl.kernel,
    out_type=(
        jax.ShapeDtypeStruct((2, NPAD, HALF), jnp.float32),
        jax.ShapeDtypeStruct((2, NTILES, NPAD), jnp.float32),
    ),
    mesh=_mesh,
    scratch_types=[
        pltpu.VMEM((BLK,), jnp.int32),
        pltpu.VMEM((BLK,), jnp.int32),
        pltpu.VMEM((SUB,), jnp.int32),
        pltpu.VMEM((SUB,), jnp.int32),
        pltpu.VMEM((SUB,), jnp.int32),
        pltpu.VMEM((SUB,), jnp.int32),
        pltpu.VMEM((SUB, HALF), jnp.float32),
        pltpu.VMEM((SUB, HALF), jnp.float32),
        pltpu.VMEM((NPAD,), jnp.float32),
        pltpu.VMEM((NPAD,), jnp.float32),
        pltpu.VMEM_SHARED((NPAD, HALF), jnp.float32),
        pltpu.SemaphoreType.DMA,
        pltpu.SemaphoreType.DMA,
    ],
    compiler_params=pltpu.CompilerParams(needs_layout_passes=False),
)
def _sc_aggregate(y2_hbm, rin_hbm, src_hbm, dst_hbm, zrows_hbm, u_hbm,
                  sparts_hbm,
                  srcb_v, dstb_v, gidx0_v, gidx1_v, sidx0_v, sidx1_v,
                  rows0_v, rows1_v, rin_v, sacc_v, uacc_sh, sem0, sem1):
    c = lax.axis_index("c")
    s = lax.axis_index("s")
    base = s * EPT

    pltpu.sync_copy(rin_hbm, rin_v)

    def zero(i, carry):
        sacc_v[pl.ds(i * 16, 16)] = jnp.zeros((16,), jnp.float32)
        return carry

    lax.fori_loop(0, NPAD // 16, zero, 0)

    pltpu.sync_copy(zrows_hbm, uacc_sh.at[pl.ds(s * ROWS_PT, ROWS_PT), :])
    plsc.subcore_barrier()

    def load_blk(b):
        pltpu.sync_copy(src_hbm.at[pl.ds(base + b * BLK, BLK)], srcb_v)
        pltpu.sync_copy(dst_hbm.at[pl.ds(base + b * BLK, BLK)], dstb_v)

    def build(m, gidx_b, sidx_b):
        off = lax.rem(m, SPB) * SUB
        for j in range(SUB // 16):
            sv = srcb_v[pl.ds(off + j * 16, 16)]
            dv = dstb_v[pl.ds(off + j * 16, 16)]
            gidx_b[pl.ds(j * 16, 16)] = sv * 2 + c
            sidx_b[pl.ds(j * 16, 16)] = dv
            rv = plsc.load_gather(rin_v, [dv])
            plsc.addupdate_scatter(sacc_v, [sv], rv)

    def gather0():
        return pltpu.async_copy(y2_hbm.at[gidx0_v], rows0_v, sem0)

    def gather1():
        return pltpu.async_copy(y2_hbm.at[gidx1_v], rows1_v, sem1)

    def wait0():
        pltpu.make_async_copy(y2_hbm.at[gidx0_v], rows0_v, sem0).wait()

    def wait1():
        pltpu.make_async_copy(y2_hbm.at[gidx1_v], rows1_v, sem1).wait()

    load_blk(0)
    build(0, gidx0_v, sidx0_v)
    gather0()

    def step(i, carry):
        k = 2 * i
        build(k + 1, gidx1_v, sidx1_v)
        gather1()
        wait0()
        pltpu.sync_copy(rows0_v, uacc_sh.at[sidx0_v], add=True)

        @pl.when(lax.rem(k + 2, SPB) == 0)
        def _refresh():
            load_blk((k + 2) // SPB)

        build(k + 2, gidx0_v, sidx0_v)
        gather0()
        wait1()
        pltpu.sync_copy(rows1_v, uacc_sh.at[sidx1_v], add=True)
        return carry

    lax.fori_loop(0, (NSUB - 1) // 2, step, 0)

    wait0()
    pltpu.sync_copy(rows0_v, uacc_sh.at[sidx0_v], add=True)

    plsc.subcore_barrier()
    pltpu.sync_copy(uacc_sh.at[pl.ds(s * ROWS_PT, ROWS_PT), :],
                    u_hbm.at[c, pl.ds(s * ROWS_PT, ROWS_PT), :])
    pltpu.sync_copy(sacc_v, sparts_hbm.at[c, s])


def _tc_final_body(u_ref, rin_ref, rout_ref, sparts_ref, b1_ref, w2_ref,
                   b2_ref, out_ref, acc_ref):
    i = pl.program_id(0)

    @pl.when(i == 0)
    def _init():
        acc_ref[...] = jnp.zeros_like(acc_ref)

    rin = rin_ref[...]
    svals = jnp.sum(sparts_ref[...], axis=(0, 1)) * 0.5
    cw = rout_ref[...] * svals
    b1 = b1_ref[...]
    ub = u_ref[...]
    h0 = jnp.maximum(rin[:, None] * ub[0] + b1[:, :HALF], 0.0)
    h1 = jnp.maximum(rin[:, None] * ub[1] + b1[:, HALF:], 0.0)
    t0 = jnp.dot(cw[None, :], h0, preferred_element_type=jnp.float32)
    t1 = jnp.dot(cw[None, :], h1, preferred_element_type=jnp.float32)
    acc_ref[:, :HALF] += t0
    acc_ref[:, HALF:] += t1

    @pl.when(i == pl.num_programs(0) - 1)
    def _fin():
        t = acc_ref[...]
        out_ref[...] = (jnp.dot(t, w2_ref[...],
                                preferred_element_type=jnp.float32) / N
                        + b2_ref[...])


def _tc_final(u, rin, rout, sparts, b1, W2, b2):
    BN = 512
    grid = (NPAD // BN,)
    nc = W2.shape[1]
    return pl.pallas_call(
        _tc_final_body,
        grid=grid,
        in_specs=[
            pl.BlockSpec((2, BN, HALF), lambda i: (0, i, 0)),
            pl.BlockSpec((BN,), lambda i: (i,)),
            pl.BlockSpec((BN,), lambda i: (i,)),
            pl.BlockSpec((2, NTILES, BN), lambda i: (0, 0, i)),
            pl.BlockSpec((1, F), lambda i: (0, 0)),
            pl.BlockSpec((F, nc), lambda i: (0, 0)),
            pl.BlockSpec((1, nc), lambda i: (0, 0)),
        ],
        out_specs=pl.BlockSpec((1, nc), lambda i: (0, 0)),
        out_shape=jax.ShapeDtypeStruct((1, nc), jnp.float32),
        scratch_shapes=[pltpu.VMEM((1, F), jnp.float32)],
    )(u, rin, rout, sparts, b1, W2, b2)


def kernel(in_feat, edge_index, W1, b1, W2, b2):
    x_pad = jnp.zeros((NPAD, F), jnp.float32).at[:N].set(in_feat)
    src = edge_index[0].astype(jnp.int32)
    dst = edge_index[1].astype(jnp.int32)

    deg_parts = _sc_degrees(src, dst)
    y, rin, rout = _tc_scale_matmul(deg_parts, x_pad, W1)
    y2 = y.reshape(2 * NPAD, HALF)
    zrows = jnp.zeros((ROWS_PT, HALF), jnp.float32)
    u, sparts = _sc_aggregate(y2, rin, src, dst, zrows)
    return _tc_final(u, rin, rout, sparts, b1.reshape(1, F), W2,
                     b2.reshape(1, -1))


# speedup vs baseline: 12.0250x; 1.0638x over previous
import functools

import jax
import jax.numpy as jnp
from jax import lax
from jax.experimental import pallas as pl
from jax.experimental.pallas import tpu as pltpu
from jax.experimental.pallas import tpu_sc as plsc

N = 10000
E = 160000
NPAD = 10240
F = 256
HALF = 128
NTILES = 16
EPT = E // NTILES
SUB = 80
NSUB = EPT // SUB
BLK = 1600
SPB = BLK // SUB
ROWS_PT = NPAD // NTILES

_mesh = plsc.VectorSubcoreMesh(core_axis_name="c", subcore_axis_name="s")


@functools.partial(
    pl.kernel,
    out_type=jax.ShapeDtypeStruct((2, NTILES, NPAD), jnp.float32),
    mesh=_mesh,
    scratch_types=[
        pltpu.VMEM((EPT,), jnp.int32),
        pltpu.VMEM((NPAD,), jnp.float32),
    ],
    compiler_params=pltpu.CompilerParams(needs_layout_passes=False),
)
def _sc_degrees(src_hbm, dst_hbm, out_hbm, idx_v, acc_v):
    c = lax.axis_index("c")
    s = lax.axis_index("s")

    @pl.when(c == 0)
    def _load_src():
        pltpu.sync_copy(src_hbm.at[pl.ds(s * EPT, EPT)], idx_v)

    @pl.when(c == 1)
    def _load_dst():
        pltpu.sync_copy(dst_hbm.at[pl.ds(s * EPT, EPT)], idx_v)

    def zero(i, carry):
        acc_v[pl.ds(i * 16, 16)] = jnp.zeros((16,), jnp.float32)
        return carry

    lax.fori_loop(0, NPAD // 16, zero, 0)

    ones = jnp.ones((16,), jnp.float32)

    def body(i, carry):
        u = idx_v[pl.ds(i * 16, 16)]
        plsc.addupdate_scatter(acc_v, [u], ones)
        return carry

    lax.fori_loop(0, EPT // 16, body, 0)
    pltpu.sync_copy(acc_v, out_hbm.at[c, s])


def _tc_scale_matmul_body(parts_ref, x_ref, w1_ref, y_ref, rin_ref, rout_ref):
    deg = jnp.sum(parts_ref[...], axis=1)
    r = lax.rsqrt(jnp.clip(deg, 1.0, None))
    rout = r[0]
    rin = r[1]
    y = jnp.dot(x_ref[...], w1_ref[...], preferred_element_type=jnp.float32)
    y = y * rout[:, None]
    y_ref[0] = y[:, :HALF]
    y_ref[1] = y[:, HALF:]
    rin_ref[...] = rin
    rout_ref[...] = rout


def _tc_scale_matmul(parts, x, W1):
    BN = 512
    grid = (NPAD // BN,)
    return pl.pallas_call(
        _tc_scale_matmul_body,
        grid=grid,
        in_specs=[
            pl.BlockSpec((2, NTILES, BN), lambda i: (0, 0, i)),
            pl.BlockSpec((BN, F), lambda i: (i, 0)),
            pl.BlockSpec((F, F), lambda i: (0, 0)),
        ],
        out_specs=[
            pl.BlockSpec((2, BN, HALF), lambda i: (0, i, 0)),
            pl.BlockSpec((BN,), lambda i: (i,)),
            pl.BlockSpec((BN,), lambda i: (i,)),
        ],
        out_shape=[
            jax.ShapeDtypeStruct((2, NPAD, HALF), jnp.float32),
            jax.ShapeDtypeStruct((NPAD,), jnp.float32),
            jax.ShapeDtypeStruct((NPAD,), jnp.float32),
        ],
    )(parts, x, W1)


@functools.partial(
    pl.kernel,
    out_type=(
        jax.ShapeDtypeStruct((2, NPAD, HALF), jnp.float32),
        jax.ShapeDtypeStruct((2, NTILES, NPAD), jnp.float32),
    ),
    mesh=_mesh,
    scratch_types=[
        pltpu.VMEM((BLK,), jnp.int32),
        pltpu.VMEM((BLK,), jnp.int32),
        pltpu.VMEM((SUB,), jnp.int32),
        pltpu.VMEM((SUB,), jnp.int32),
        pltpu.VMEM((SUB,), jnp.int32),
        pltpu.VMEM((SUB,), jnp.int32),
        pltpu.VMEM((SUB, HALF), jnp.float32),
        pltpu.VMEM((SUB, HALF), jnp.float32),
        pltpu.VMEM((NPAD,), jnp.float32),
        pltpu.VMEM((NPAD,), jnp.float32),
        pltpu.VMEM_SHARED((NPAD, HALF), jnp.float32),
        pltpu.SemaphoreType.DMA,
        pltpu.SemaphoreType.DMA,
        pltpu.SemaphoreType.DMA,
        pltpu.SemaphoreType.DMA,
    ],
    compiler_params=pltpu.CompilerParams(needs_layout_passes=False),
)
def _sc_aggregate(y2_hbm, rin_hbm, src_hbm, dst_hbm, zrows_hbm, u_hbm,
                  sparts_hbm,
                  srcb_v, dstb_v, gidx0_v, gidx1_v, sidx0_v, sidx1_v,
                  rows0_v, rows1_v, rin_v, sacc_v, uacc_sh,
                  sem0, sem1, ssem0, ssem1):
    c = lax.axis_index("c")
    s = lax.axis_index("s")
    base = s * EPT
    cbase = c * NPAD

    pltpu.sync_copy(rin_hbm, rin_v)

    def zero(i, carry):
        sacc_v[pl.ds(i * 16, 16)] = jnp.zeros((16,), jnp.float32)
        return carry

    lax.fori_loop(0, NPAD // 16, zero, 0)

    pltpu.sync_copy(zrows_hbm, uacc_sh.at[pl.ds(s * ROWS_PT, ROWS_PT), :])
    plsc.subcore_barrier()

    def load_blk(b):
        pltpu.sync_copy(src_hbm.at[pl.ds(base + b * BLK, BLK)], srcb_v)
        pltpu.sync_copy(dst_hbm.at[pl.ds(base + b * BLK, BLK)], dstb_v)

    def build(m, gidx_b, sidx_b):
        off = lax.rem(m, SPB) * SUB
        for j in range(SUB // 16):
            sv = srcb_v[pl.ds(off + j * 16, 16)]
            dv = dstb_v[pl.ds(off + j * 16, 16)]
            gidx_b[pl.ds(j * 16, 16)] = sv + cbase
            sidx_b[pl.ds(j * 16, 16)] = dv
            rv = plsc.load_gather(rin_v, [dv])
            plsc.addupdate_scatter(sacc_v, [sv], rv)

    def gather0():
        pltpu.async_copy(y2_hbm.at[gidx0_v], rows0_v, sem0)

    def gather1():
        pltpu.async_copy(y2_hbm.at[gidx1_v], rows1_v, sem1)

    def wait_g0():
        pltpu.make_async_copy(y2_hbm.at[gidx0_v], rows0_v, sem0).wait()

    def wait_g1():
        pltpu.make_async_copy(y2_hbm.at[gidx1_v], rows1_v, sem1).wait()

    def scatter0():
        pltpu.async_copy(rows0_v, uacc_sh.at[sidx0_v], ssem0, add=True)

    def scatter1():
        pltpu.async_copy(rows1_v, uacc_sh.at[sidx1_v], ssem1, add=True)

    def wait_s0():
        pltpu.make_async_copy(rows0_v, uacc_sh.at[sidx0_v], ssem0).wait()

    def wait_s1():
        pltpu.make_async_copy(rows1_v, uacc_sh.at[sidx1_v], ssem1).wait()

    load_blk(0)
    build(0, gidx0_v, sidx0_v)
    gather0()
    build(1, gidx1_v, sidx1_v)
    gather1()
    wait_g0()
    scatter0()
    wait_s0()
    build(2, gidx0_v, sidx0_v)
    gather0()
    wait_g1()
    scatter1()

    def step(i, carry):
        k = 2 * i
        wait_s1()
        build(k + 1, gidx1_v, sidx1_v)
        gather1()
        wait_g0()
        scatter0()
        wait_s0()

        @pl.when(lax.rem(k + 2, SPB) == 0)
        def _refresh():
            load_blk((k + 2) // SPB)

        build(k + 2, gidx0_v, sidx0_v)
        gather0()
        wait_g1()
        scatter1()
        return carry

    lax.fori_loop(1, (NSUB - 1) // 2, step, 0)

    wait_s1()
    wait_g0()
    pltpu.sync_copy(rows0_v, uacc_sh.at[sidx0_v], add=True)

    plsc.subcore_barrier()
    pltpu.sync_copy(uacc_sh.at[pl.ds(s * ROWS_PT, ROWS_PT), :],
                    u_hbm.at[c, pl.ds(s * ROWS_PT, ROWS_PT), :])
    pltpu.sync_copy(sacc_v, sparts_hbm.at[c, s])


def _tc_final_body(u_ref, rin_ref, rout_ref, sparts_ref, b1_ref, w2_ref,
                   b2_ref, out_ref, acc_ref):
    i = pl.program_id(0)

    @pl.when(i == 0)
    def _init():
        acc_ref[...] = jnp.zeros_like(acc_ref)

    rin = rin_ref[...]
    svals = jnp.sum(sparts_ref[...], axis=(0, 1)) * 0.5
    cw = rout_ref[...] * svals
    b1 = b1_ref[...]
    ub = u_ref[...]
    h0 = jnp.maximum(rin[:, None] * ub[0] + b1[:, :HALF], 0.0)
    h1 = jnp.maximum(rin[:, None] * ub[1] + b1[:, HALF:], 0.0)
    t0 = jnp.dot(cw[None, :], h0, preferred_element_type=jnp.float32)
    t1 = jnp.dot(cw[None, :], h1, preferred_element_type=jnp.float32)
    acc_ref[:, :HALF] += t0
    acc_ref[:, HALF:] += t1

    @pl.when(i == pl.num_programs(0) - 1)
    def _fin():
        t = acc_ref[...]
        out_ref[...] = (jnp.dot(t, w2_ref[...],
                                preferred_element_type=jnp.float32) / N
                        + b2_ref[...])


def _tc_final(u, rin, rout, sparts, b1, W2, b2):
    BN = 512
    grid = (NPAD // BN,)
    nc = W2.shape[1]
    return pl.pallas_call(
        _tc_final_body,
        grid=grid,
        in_specs=[
            pl.BlockSpec((2, BN, HALF), lambda i: (0, i, 0)),
            pl.BlockSpec((BN,), lambda i: (i,)),
            pl.BlockSpec((BN,), lambda i: (i,)),
            pl.BlockSpec((2, NTILES, BN), lambda i: (0, 0, i)),
            pl.BlockSpec((1, F), lambda i: (0, 0)),
            pl.BlockSpec((F, nc), lambda i: (0, 0)),
            pl.BlockSpec((1, nc), lambda i: (0, 0)),
        ],
        out_specs=pl.BlockSpec((1, nc), lambda i: (0, 0)),
        out_shape=jax.ShapeDtypeStruct((1, nc), jnp.float32),
        scratch_shapes=[pltpu.VMEM((1, F), jnp.float32)],
    )(u, rin, rout, sparts, b1, W2, b2)


def kernel(in_feat, edge_index, W1, b1, W2, b2):
    src = edge_index[0].astype(jnp.int32)
    dst = edge_index[1].astype(jnp.int32)

    deg_parts = _sc_degrees(src, dst)
    y, rin, rout = _tc_scale_matmul(deg_parts, in_feat, W1)
    y2 = y.reshape(2 * NPAD, HALF)
    zrows = jnp.zeros((ROWS_PT, HALF), jnp.float32)
    u, sparts = _sc_aggregate(y2, rin, src, dst, zrows)
    return _tc_final(u, rin, rout, sparts, b1.reshape(1, F), W2,
                     b2.reshape(1, -1))
